# baseline (speedup 1.0000x reference)
import functools

import jax
import jax.numpy as jnp
from jax.experimental import pallas as pl
from jax.experimental.pallas import tpu as pltpu

DIM = 1024
HID = 4096
E = 8
K = 2
R = 16
S = 2048
NA = S * K
T = 256
J = 8
HT = HID // J
NSTEPS = NA // T + E - 1
YROWS = (NSTEPS + 1) * T
CH = 512


def _router_kernel(x_ref, gw_ref, ew_ref, pos_ref, meta_ref):
    x = x_ref[...]
    gw = gw_ref[...]
    logits = jax.lax.dot_general(x, gw, (((1,), (1,)), ((), ())),
                                 preferred_element_type=jnp.float32)
    lane = jax.lax.broadcasted_iota(jnp.int32, (S, 128), 1)
    neg = jnp.float32(-1e30)
    logits = jnp.where(lane < E, logits, neg)
    m0 = jnp.max(logits, axis=1, keepdims=True)
    i0 = jnp.min(jnp.where(logits == m0, lane, 128), axis=1, keepdims=True)
    masked = jnp.where(lane == i0, neg, logits)
    m1 = jnp.max(masked, axis=1, keepdims=True)
    i1 = jnp.min(jnp.where(masked == m1, lane, 128), axis=1, keepdims=True)
    w0 = 1.0 / (1.0 + jnp.exp(m1 - m0))
    w1v = 1.0 - w0

    oh0 = (lane == i0).astype(jnp.float32)
    oh1 = (lane == i1).astype(jnp.float32)
    a = oh0 + oh1

    r = jax.lax.broadcasted_iota(jnp.int32, (CH, CH), 0)
    c = jax.lax.broadcasted_iota(jnp.int32, (CH, CH), 1)
    tri = (r >= c).astype(jnp.float32)
    carry = jnp.zeros((1, 128), jnp.float32)
    chunks = []
    for k in range(S // CH):
        blk = a[k * CH:(k + 1) * CH, :]
        incl = jax.lax.dot_general(tri, blk, (((1,), (0,)), ((), ())),
                                   preferred_element_type=jnp.float32)
        chunks.append(incl + carry)
        carry = carry + incl[CH - 1:CH, :]
    s_incl = jnp.concatenate(chunks, axis=0)
    s_excl = s_incl - a
    counts = carry

    ntile = jnp.floor((counts + (T - 1)) * (1.0 / T))
    lr = jax.lax.broadcasted_iota(jnp.int32, (128, 128), 0)
    lc = jax.lax.broadcasted_iota(jnp.int32, (128, 128), 1)
    mle = (lr <= lc).astype(jnp.float32)
    cum_incl = jax.lax.dot_general(ntile, mle, (((1,), (0,)), ((), ())),
                                   preferred_element_type=jnp.float32)
    cum_excl = cum_incl - ntile
    ntot = jnp.sum(ntile, axis=1, keepdims=True)
    off = cum_excl * T

    pos0 = jnp.sum(oh0 * (off + s_excl), axis=1, keepdims=True)
    pos1 = jnp.sum(oh1 * (off + s_excl), axis=1, keepdims=True)
    ew_ref[...] = jnp.where(lane == 0, w0, jnp.where(lane == 1, w1v, 0.0))
    posf = jnp.where(lane == 0, pos0, jnp.where(lane == 1, pos1, 0.0))
    pos_ref[...] = posf.astype(jnp.int32)

    ident = (lr == lc).astype(jnp.float32)
    ccol = jax.lax.dot_general(ident, cum_incl, (((1,), (1,)), ((), ())),
                               preferred_element_type=jnp.float32)
    mrc = ((ccol <= lc.astype(jnp.float32)) & (lr < E)).astype(jnp.float32)
    step_e = jnp.minimum(jnp.sum(mrc, axis=0, keepdims=True), E - 1)
    meta = jnp.where(lane[:1, :] == 31, ntot, step_e)
    meta_ref[...] = jnp.broadcast_to(meta, (8, 128)).astype(jnp.int32)


def _dispatch_kernel(pos_ref, x_ref, out_ref):
    out_ref[...] = x_ref[...]


def _combine_kernel(pos_ref, r0_ref, r1_ref, ew_ref, out_ref):
    w0 = ew_ref[0, 0, 0]
    w1v = ew_ref[0, 0, 1]
    out_ref[...] = w0 * r0_ref[...] + w1v * r1_ref[...]


def _ffn_kernel(se_ref, nt_ref, xs_ref, w1_ref, b1_ref, a1_ref, l1_ref,
                w3_ref, b3_ref, a3_ref, l3_ref, w2_ref, b2_ref, a2_ref, l2_ref,
                out_ref, yacc_ref, racc_ref):
    j = pl.program_id(0)
    g = pl.program_id(1)
    ntot = nt_ref[0]

    @pl.when(g < ntot)
    def _():
        xt = xs_ref[...].astype(jnp.bfloat16)
        w1b = w1_ref[0].astype(jnp.bfloat16)
        w3b = w3_ref[0].astype(jnp.bfloat16)
        a1b = a1_ref[0].astype(jnp.bfloat16)
        a3b = a3_ref[0].astype(jnp.bfloat16)
        l1b = l1_ref[0].astype(jnp.bfloat16)
        l3b = l3_ref[0].astype(jnp.bfloat16)

        dn = (((1,), (1,)), ((), ()))
        xw1 = jax.lax.dot_general(xt, w1b, dn, preferred_element_type=jnp.float32)
        xa1 = jax.lax.dot_general(xt, a1b, dn, preferred_element_type=jnp.float32)
        u = xw1 + jax.lax.dot_general(xa1.astype(jnp.bfloat16), l1b, dn,
                                      preferred_element_type=jnp.float32)
        u = u + b1_ref[0]
        xw3 = jax.lax.dot_general(xt, w3b, dn, preferred_element_type=jnp.float32)
        xa3 = jax.lax.dot_general(xt, a3b, dn, preferred_element_type=jnp.float32)
        v = xw3 + jax.lax.dot_general(xa3.astype(jnp.bfloat16), l3b, dn,
                                      preferred_element_type=jnp.float32)
        v = v + b3_ref[0]
        h = (u * (1.0 / (1.0 + jnp.exp(-u))) * v).astype(jnp.bfloat16)

        w2b = w2_ref[0].astype(jnp.bfloat16)
        a2b = a2_ref[0].astype(jnp.bfloat16)
        dy = jax.lax.dot_general(h, w2b, dn, preferred_element_type=jnp.float32)
        dr = jax.lax.dot_general(h, a2b, dn, preferred_element_type=jnp.float32)

        roff = pl.multiple_of(g * T, T)

        @pl.when(j == 0)
        def _():
            yacc_ref[pl.ds(roff, T), :] = dy
            racc_ref[pl.ds(roff, T), :] = dr

        @pl.when(j > 0)
        def _():
            yacc_ref[pl.ds(roff, T), :] += dy
            racc_ref[pl.ds(roff, T), :] += dr

        @pl.when(j == J - 1)
        def _():
            rk = racc_ref[pl.ds(roff, T), :].astype(jnp.bfloat16)
            l2b = l2_ref[0].astype(jnp.bfloat16)
            y = yacc_ref[pl.ds(roff, T), :]
            y = y + jax.lax.dot_general(rk, l2b, dn,
                                        preferred_element_type=jnp.float32)
            out_ref[...] = y + b2_ref[0]


def kernel(x, gate_w, w1, b1, a1, l1, w2, b2, a2, l2, w3, b3, a3, l3):
    orig_shape = x.shape
    xf = x.reshape(-1, DIM)
    gwp = jnp.pad(gate_w, ((0, 128 - E), (0, 0)))

    ew128, pos128, meta = pl.pallas_call(
        _router_kernel,
        out_shape=[
            jax.ShapeDtypeStruct((S, 128), jnp.float32),
            jax.ShapeDtypeStruct((S, 128), jnp.int32),
            jax.ShapeDtypeStruct((8, 128), jnp.int32),
        ],
    )(xf, gwp)

    pos = pos128[:, :K].reshape(NA)
    step_e = meta[0, :NSTEPS]
    ntot = meta[0, 31:32]

    xs3 = pl.pallas_call(
        _dispatch_kernel,
        grid_spec=pltpu.PrefetchScalarGridSpec(
            num_scalar_prefetch=1,
            grid=(NA,),
            in_specs=[pl.BlockSpec((1, 1, DIM), lambda i, pos_r: (i // K, 0, 0))],
            out_specs=pl.BlockSpec((1, 1, DIM), lambda i, pos_r: (pos_r[i], 0, 0)),
        ),
        out_shape=jax.ShapeDtypeStruct((YROWS, 1, DIM), jnp.float32),
    )(pos, xf.reshape(S, 1, DIM))
    xs = xs3.reshape(YROWS, DIM)

    def _ge(g, nt_r):
        return jnp.minimum(g, nt_r[0] - 1)

    ys = pl.pallas_call(
        _ffn_kernel,
        grid_spec=pltpu.PrefetchScalarGridSpec(
            num_scalar_prefetch=2,
            grid=(J, NSTEPS),
            in_specs=[
                pl.BlockSpec((T, DIM), lambda j, g, se, nt: (_ge(g, nt), 0)),
                pl.BlockSpec((1, HT, DIM), lambda j, g, se, nt: (se[_ge(g, nt)], j, 0)),
                pl.BlockSpec((1, 1, HT), lambda j, g, se, nt: (se[_ge(g, nt)], 0, j)),
                pl.BlockSpec((1, R, DIM), lambda j, g, se, nt: (se[_ge(g, nt)], 0, 0)),
                pl.BlockSpec((1, HT, R), lambda j, g, se, nt: (se[_ge(g, nt)], j, 0)),
                pl.BlockSpec((1, HT, DIM), lambda j, g, se, nt: (se[_ge(g, nt)], j, 0)),
                pl.BlockSpec((1, 1, HT), lambda j, g, se, nt: (se[_ge(g, nt)], 0, j)),
                pl.BlockSpec((1, R, DIM), lambda j, g, se, nt: (se[_ge(g, nt)], 0, 0)),
                pl.BlockSpec((1, HT, R), lambda j, g, se, nt: (se[_ge(g, nt)], j, 0)),
                pl.BlockSpec((1, DIM, HT), lambda j, g, se, nt: (se[_ge(g, nt)], 0, j)),
                pl.BlockSpec((1, 1, DIM), lambda j, g, se, nt: (se[_ge(g, nt)], 0, 0)),
                pl.BlockSpec((1, R, HT), lambda j, g, se, nt: (se[_ge(g, nt)], 0, j)),
                pl.BlockSpec((1, DIM, R), lambda j, g, se, nt: (se[_ge(g, nt)], 0, 0)),
            ],
            out_specs=pl.BlockSpec(
                (T, DIM),
                lambda j, g, se, nt: (jnp.where(j == J - 1, _ge(g, nt), NSTEPS), 0)),
            scratch_shapes=[
                pltpu.VMEM((NSTEPS * T, DIM), jnp.float32),
                pltpu.VMEM((NSTEPS * T, R), jnp.float32),
            ],
        ),
        out_shape=jax.ShapeDtypeStruct((YROWS, DIM), jnp.float32),
    )(step_e, ntot, xs, w1, b1.reshape(E, 1, HID), a1, l1,
      w3, b3.reshape(E, 1, HID), a3, l3, w2, b2.reshape(E, 1, DIM), a2, l2)
    ys3 = ys.reshape(YROWS, 1, DIM)

    y = pl.pallas_call(
        _combine_kernel,
        grid_spec=pltpu.PrefetchScalarGridSpec(
            num_scalar_prefetch=1,
            grid=(S,),
            in_specs=[
                pl.BlockSpec((1, 1, DIM), lambda t, pos_r: (pos_r[K * t], 0, 0)),
                pl.BlockSpec((1, 1, DIM), lambda t, pos_r: (pos_r[K * t + 1], 0, 0)),
                pl.BlockSpec((1, 1, 128), lambda t, pos_r: (t, 0, 0)),
            ],
            out_specs=pl.BlockSpec((1, 1, DIM), lambda t, pos_r: (t, 0, 0)),
        ),
        out_shape=jax.ShapeDtypeStruct((S, 1, DIM), jnp.float32),
    )(pos, ys3, ys3, ew128.reshape(S, 1, 128))

    return y.reshape(orig_shape)


# speedup vs baseline: 5.2901x; 5.2901x over previous
import functools

import jax
import jax.numpy as jnp
from jax import lax
from jax.experimental import pallas as pl
from jax.experimental.pallas import tpu as pltpu
from jax.experimental.pallas import tpu_sc as plsc

DIM = 1024
HID = 4096
E = 8
K = 2
R = 16
S = 2048
NA = S * K
T = 256
J = 8
HT = HID // J
NSTEPS = NA // T + E - 1
YROWS = (NSTEPS + 1) * T
CH = 512


def _router_kernel(x_ref, gw_ref, ew_ref, pos_ref, meta_ref):
    x = x_ref[...]
    gw = gw_ref[...]
    logits = jax.lax.dot_general(x, gw, (((1,), (1,)), ((), ())),
                                 preferred_element_type=jnp.float32)
    lane = jax.lax.broadcasted_iota(jnp.int32, (S, 128), 1)
    neg = jnp.float32(-1e30)
    logits = jnp.where(lane < E, logits, neg)
    m0 = jnp.max(logits, axis=1, keepdims=True)
    i0 = jnp.min(jnp.where(logits == m0, lane, 128), axis=1, keepdims=True)
    masked = jnp.where(lane == i0, neg, logits)
    m1 = jnp.max(masked, axis=1, keepdims=True)
    i1 = jnp.min(jnp.where(masked == m1, lane, 128), axis=1, keepdims=True)
    w0 = 1.0 / (1.0 + jnp.exp(m1 - m0))
    w1v = 1.0 - w0

    oh0 = (lane == i0).astype(jnp.float32)
    oh1 = (lane == i1).astype(jnp.float32)
    a = oh0 + oh1

    r = jax.lax.broadcasted_iota(jnp.int32, (CH, CH), 0)
    c = jax.lax.broadcasted_iota(jnp.int32, (CH, CH), 1)
    tri = (r >= c).astype(jnp.float32)
    carry = jnp.zeros((1, 128), jnp.float32)
    chunks = []
    for k in range(S // CH):
        blk = a[k * CH:(k + 1) * CH, :]
        incl = jax.lax.dot_general(tri, blk, (((1,), (0,)), ((), ())),
                                   preferred_element_type=jnp.float32)
        chunks.append(incl + carry)
        carry = carry + incl[CH - 1:CH, :]
    s_incl = jnp.concatenate(chunks, axis=0)
    s_excl = s_incl - a
    counts = carry

    ntile = jnp.floor((counts + (T - 1)) * (1.0 / T))
    lr = jax.lax.broadcasted_iota(jnp.int32, (128, 128), 0)
    lc = jax.lax.broadcasted_iota(jnp.int32, (128, 128), 1)
    mle = (lr <= lc).astype(jnp.float32)
    cum_incl = jax.lax.dot_general(ntile, mle, (((1,), (0,)), ((), ())),
                                   preferred_element_type=jnp.float32)
    cum_excl = cum_incl - ntile
    ntot = jnp.sum(ntile, axis=1, keepdims=True)
    off = cum_excl * T

    pos0 = jnp.sum(oh0 * (off + s_excl), axis=1, keepdims=True)
    pos1 = jnp.sum(oh1 * (off + s_excl), axis=1, keepdims=True)
    ew_ref[...] = jnp.where(lane == 0, w0, jnp.where(lane == 1, w1v, 0.0))
    posf = jnp.where(lane == 0, pos0, jnp.where(lane == 1, pos1, 0.0))
    pos_ref[...] = posf.astype(jnp.int32)

    ident = (lr == lc).astype(jnp.float32)
    ccol = jax.lax.dot_general(ident, cum_incl, (((1,), (1,)), ((), ())),
                               preferred_element_type=jnp.float32)
    mrc = ((ccol <= lc.astype(jnp.float32)) & (lr < E)).astype(jnp.float32)
    step_e = jnp.minimum(jnp.sum(mrc, axis=0, keepdims=True), E - 1)
    meta = jnp.where(lane[:1, :] == 31, ntot, step_e)
    meta_ref[...] = jnp.broadcast_to(meta, (8, 128)).astype(jnp.int32)


NC = 2
NS = 16
NW = NC * NS
APW = NA // NW
RB = 64
_SC_MESH = plsc.VectorSubcoreMesh(core_axis_name="c", subcore_axis_name="s")


def _sc_dispatch(x_hbm, pos_hbm, xs_hbm, idx_a, idx_b, src_v, rows_v, sem):
    wid = lax.axis_index("s") * NC + lax.axis_index("c")
    for b, idx_v in ((0, idx_a), (1, idx_b)):
        base = wid * APW + b * RB
        pltpu.sync_copy(pos_hbm.at[pl.ds(base, RB)], idx_v)
        for m in range(RB // 16):
            src_v[pl.ds(16 * m, 16)] = jnp.right_shift(
                lax.iota(jnp.int32, 16) + (base + 16 * m), 1)
        pltpu.async_copy(x_hbm.at[src_v], rows_v, sem).wait()
        pltpu.async_copy(rows_v, xs_hbm.at[idx_v], sem).wait()


def _sc_gather(ys_hbm, pos_hbm, out_hbm, idx_v, rows_v, sem):
    wid = lax.axis_index("s") * NC + lax.axis_index("c")
    for b in range(2):
        base = wid * APW + b * RB
        pltpu.sync_copy(pos_hbm.at[pl.ds(base, RB)], idx_v)
        pltpu.async_copy(ys_hbm.at[idx_v], rows_v, sem).wait()
        pltpu.sync_copy(rows_v, out_hbm.at[pl.ds(base, RB), :])


def _wcombine_kernel(yg_ref, ew_ref, out_ref):
    blk = yg_ref[...]
    w0 = ew_ref[:, 0:1]
    w1 = ew_ref[:, 1:2]
    out_ref[...] = w0 * blk[:, :DIM] + w1 * blk[:, DIM:]


def _ffn_kernel(se_ref, nt_ref, xs_ref, w1_ref, b1_ref, a1_ref, l1_ref,
                w3_ref, b3_ref, a3_ref, l3_ref, w2_ref, b2_ref, a2_ref, l2_ref,
                out_ref, yacc_ref, racc_ref):
    j = pl.program_id(0)
    g = pl.program_id(1)
    ntot = nt_ref[0]

    @pl.when(g < ntot)
    def _():
        xt = xs_ref[...].astype(jnp.bfloat16)
        w1b = w1_ref[0].astype(jnp.bfloat16)
        w3b = w3_ref[0].astype(jnp.bfloat16)
        a1b = a1_ref[0].astype(jnp.bfloat16)
        a3b = a3_ref[0].astype(jnp.bfloat16)
        l1b = l1_ref[0].astype(jnp.bfloat16)
        l3b = l3_ref[0].astype(jnp.bfloat16)

        dn = (((1,), (1,)), ((), ()))
        xw1 = jax.lax.dot_general(xt, w1b, dn, preferred_element_type=jnp.float32)
        xa1 = jax.lax.dot_general(xt, a1b, dn, preferred_element_type=jnp.float32)
        u = xw1 + jax.lax.dot_general(xa1.astype(jnp.bfloat16), l1b, dn,
                                      preferred_element_type=jnp.float32)
        u = u + b1_ref[0]
        xw3 = jax.lax.dot_general(xt, w3b, dn, preferred_element_type=jnp.float32)
        xa3 = jax.lax.dot_general(xt, a3b, dn, preferred_element_type=jnp.float32)
        v = xw3 + jax.lax.dot_general(xa3.astype(jnp.bfloat16), l3b, dn,
                                      preferred_element_type=jnp.float32)
        v = v + b3_ref[0]
        h = (u * (1.0 / (1.0 + jnp.exp(-u))) * v).astype(jnp.bfloat16)

        w2b = w2_ref[0].astype(jnp.bfloat16)
        a2b = a2_ref[0].astype(jnp.bfloat16)
        dy = jax.lax.dot_general(h, w2b, dn, preferred_element_type=jnp.float32)
        dr = jax.lax.dot_general(h, a2b, dn, preferred_element_type=jnp.float32)

        roff = pl.multiple_of(g * T, T)

        @pl.when(j == 0)
        def _():
            yacc_ref[pl.ds(roff, T), :] = dy
            racc_ref[pl.ds(roff, T), :] = dr

        @pl.when(j > 0)
        def _():
            yacc_ref[pl.ds(roff, T), :] += dy
            racc_ref[pl.ds(roff, T), :] += dr

        @pl.when(j == J - 1)
        def _():
            rk = racc_ref[pl.ds(roff, T), :].astype(jnp.bfloat16)
            l2b = l2_ref[0].astype(jnp.bfloat16)
            y = yacc_ref[pl.ds(roff, T), :]
            y = y + jax.lax.dot_general(rk, l2b, dn,
                                        preferred_element_type=jnp.float32)
            out_ref[...] = y + b2_ref[0]


def kernel(x, gate_w, w1, b1, a1, l1, w2, b2, a2, l2, w3, b3, a3, l3):
    orig_shape = x.shape
    xf = x.reshape(-1, DIM)
    gwp = jnp.pad(gate_w, ((0, 128 - E), (0, 0)))

    ew128, pos128, meta = pl.pallas_call(
        _router_kernel,
        out_shape=[
            jax.ShapeDtypeStruct((S, 128), jnp.float32),
            jax.ShapeDtypeStruct((S, 128), jnp.int32),
            jax.ShapeDtypeStruct((8, 128), jnp.int32),
        ],
    )(xf, gwp)

    pos = pos128[:, :K].reshape(NA)
    step_e = meta[0, :NSTEPS]
    ntot = meta[0, 31:32]

    ewf = ew128[:, :K].reshape(NA)

    dispatch = functools.partial(
        pl.kernel,
        out_type=jax.ShapeDtypeStruct((YROWS, DIM), jnp.float32),
        scratch_types=[
            pltpu.VMEM((RB,), jnp.int32),
            pltpu.VMEM((RB,), jnp.int32),
            pltpu.VMEM((RB,), jnp.int32),
            pltpu.VMEM((RB, DIM), jnp.float32),
            pltpu.SemaphoreType.DMA,
        ],
        mesh=_SC_MESH,
    )(_sc_dispatch)
    xs = dispatch(xf, pos)

    def _ge(g, nt_r):
        return jnp.minimum(g, nt_r[0] - 1)

    ys = pl.pallas_call(
        _ffn_kernel,
        grid_spec=pltpu.PrefetchScalarGridSpec(
            num_scalar_prefetch=2,
            grid=(J, NSTEPS),
            in_specs=[
                pl.BlockSpec((T, DIM), lambda j, g, se, nt: (_ge(g, nt), 0)),
                pl.BlockSpec((1, HT, DIM), lambda j, g, se, nt: (se[_ge(g, nt)], j, 0)),
                pl.BlockSpec((1, 1, HT), lambda j, g, se, nt: (se[_ge(g, nt)], 0, j)),
                pl.BlockSpec((1, R, DIM), lambda j, g, se, nt: (se[_ge(g, nt)], 0, 0)),
                pl.BlockSpec((1, HT, R), lambda j, g, se, nt: (se[_ge(g, nt)], j, 0)),
                pl.BlockSpec((1, HT, DIM), lambda j, g, se, nt: (se[_ge(g, nt)], j, 0)),
                pl.BlockSpec((1, 1, HT), lambda j, g, se, nt: (se[_ge(g, nt)], 0, j)),
                pl.BlockSpec((1, R, DIM), lambda j, g, se, nt: (se[_ge(g, nt)], 0, 0)),
                pl.BlockSpec((1, HT, R), lambda j, g, se, nt: (se[_ge(g, nt)], j, 0)),
                pl.BlockSpec((1, DIM, HT), lambda j, g, se, nt: (se[_ge(g, nt)], 0, j)),
                pl.BlockSpec((1, 1, DIM), lambda j, g, se, nt: (se[_ge(g, nt)], 0, 0)),
                pl.BlockSpec((1, R, HT), lambda j, g, se, nt: (se[_ge(g, nt)], 0, j)),
                pl.BlockSpec((1, DIM, R), lambda j, g, se, nt: (se[_ge(g, nt)], 0, 0)),
            ],
            out_specs=pl.BlockSpec(
                (T, DIM),
                lambda j, g, se, nt: (jnp.where(j == J - 1, _ge(g, nt), NSTEPS), 0)),
            scratch_shapes=[
                pltpu.VMEM((NSTEPS * T, DIM), jnp.float32),
                pltpu.VMEM((NSTEPS * T, R), jnp.float32),
            ],
        ),
        out_shape=jax.ShapeDtypeStruct((YROWS, DIM), jnp.float32),
    )(step_e, ntot, xs, w1, b1.reshape(E, 1, HID), a1, l1,
      w3, b3.reshape(E, 1, HID), a3, l3, w2, b2.reshape(E, 1, DIM), a2, l2)

    gather = functools.partial(
        pl.kernel,
        out_type=jax.ShapeDtypeStruct((NA, DIM), jnp.float32),
        scratch_types=[
            pltpu.VMEM((RB,), jnp.int32),
            pltpu.VMEM((RB, DIM), jnp.float32),
            pltpu.SemaphoreType.DMA,
        ],
        mesh=_SC_MESH,
    )(_sc_gather)
    yg = gather(ys, pos).reshape(S, K * DIM)

    CT = 256
    y = pl.pallas_call(
        _wcombine_kernel,
        grid=(S // CT,),
        in_specs=[
            pl.BlockSpec((CT, K * DIM), lambda t: (t, 0)),
            pl.BlockSpec((CT, 128), lambda t: (t, 0)),
        ],
        out_specs=pl.BlockSpec((CT, DIM), lambda t: (t, 0)),
        out_shape=jax.ShapeDtypeStruct((S, DIM), jnp.float32),
    )(yg, ew128)

    return y.reshape(orig_shape)
